# baseline (speedup 1.0000x reference)
import functools
import math

import jax
import jax.numpy as jnp
from jax import lax
from jax.experimental import pallas as pl
from jax.experimental.pallas import tpu as pltpu
from jax.experimental.pallas import tpu_sc as plsc

NC = 2
NS = 16
NW = NC * NS
CHUNK = 256


def _sc_mesh():
    return plsc.VectorSubcoreMesh(core_axis_name="c", subcore_axis_name="s")


_SC_PARAMS = pltpu.CompilerParams(use_tc_tiling_on_sc=False)


def _deg_hist(col_t, n_pad, cpt):
    stripe = n_pad // NS

    @functools.partial(
        pl.kernel,
        out_type=jax.ShapeDtypeStruct((NC, n_pad), jnp.float32),
        mesh=_sc_mesh(),
        scratch_types=[
            pltpu.VMEM((cpt, CHUNK), jnp.int32),
            pltpu.VMEM((CHUNK,), jnp.float32),
            pltpu.VMEM((stripe,), jnp.float32),
            pltpu.VMEM_SHARED((n_pad,), jnp.float32),
        ],
        compiler_params=_SC_PARAMS,
    )
    def k(col_hbm, out_hbm, col_v, ones_v, zb_v, acc_sh):
        cid = lax.axis_index("c")
        sid = lax.axis_index("s")
        wid = cid * NS + sid
        for i in range(CHUNK // 16):
            ones_v[pl.ds(i * 16, 16)] = jnp.ones((16,), jnp.float32)

        def zb(i, carry):
            zb_v[pl.ds(i * 16, 16)] = jnp.zeros((16,), jnp.float32)
            return carry

        lax.fori_loop(0, stripe // 16, zb, 0)
        pltpu.sync_copy(zb_v, acc_sh.at[pl.ds(sid * stripe, stripe)])
        pltpu.sync_copy(col_hbm.at[wid], col_v)
        plsc.subcore_barrier()

        def body(j, carry):
            pltpu.sync_copy(ones_v, acc_sh.at[col_v.at[j]], add=True)
            return carry

        lax.fori_loop(0, cpt, body, 0)
        plsc.subcore_barrier()
        pltpu.sync_copy(
            acc_sh.at[pl.ds(sid * stripe, stripe)],
            out_hbm.at[cid, pl.ds(sid * stripe, stripe)],
        )

    return k(col_t)


def _prep_tc(x_pad, W1, d0, d1, n_pad, bn):
    d_in = x_pad.shape[1]
    d_hid = W1.shape[1]

    def body(x_ref, w_ref, d0_ref, d1_ref, y_ref, dis_ref):
        deg = d0_ref[...] + d1_ref[...] + 1.0
        dis = lax.rsqrt(deg)
        xw = jnp.dot(x_ref[...], w_ref[...], preferred_element_type=jnp.float32)
        y_ref[...] = xw * dis[:, None]
        dis_ref[...] = dis

    return pl.pallas_call(
        body,
        grid=(n_pad // bn,),
        in_specs=[
            pl.BlockSpec((bn, d_in), lambda i: (i, 0)),
            pl.BlockSpec((d_in, d_hid), lambda i: (0, 0)),
            pl.BlockSpec((bn,), lambda i: (i,)),
            pl.BlockSpec((bn,), lambda i: (i,)),
        ],
        out_specs=[
            pl.BlockSpec((bn, d_hid), lambda i: (i, 0)),
            pl.BlockSpec((bn,), lambda i: (i,)),
        ],
        out_shape=[
            jax.ShapeDtypeStruct((n_pad, d_hid), jnp.float32),
            jax.ShapeDtypeStruct((n_pad,), jnp.float32),
        ],
    )(x_pad, W1, d0, d1)


def _edge_pass(y_pad, dis_pad, row_t, col_t, n_pad, cpt, kdepth):
    stripe = n_pad // NS
    d_hid = y_pad.shape[1]
    qs = stripe // 16

    @functools.partial(
        pl.kernel,
        out_type=(
            jax.ShapeDtypeStruct((NC, n_pad, d_hid), jnp.float32),
            jax.ShapeDtypeStruct((NC, n_pad), jnp.float32),
        ),
        mesh=_sc_mesh(),
        scratch_types=[
            pltpu.VMEM((cpt, CHUNK), jnp.int32),
            pltpu.VMEM((cpt, CHUNK), jnp.int32),
            pltpu.VMEM((1, CHUNK, d_hid), jnp.float32),
            pltpu.VMEM((1, CHUNK), jnp.float32),
            pltpu.VMEM((qs, d_hid), jnp.float32),
            pltpu.VMEM((stripe,), jnp.float32),
            pltpu.SemaphoreType.DMA((2, kdepth)),
            pltpu.SemaphoreType.DMA((2, kdepth)),
            pltpu.VMEM_SHARED((n_pad, d_hid), jnp.float32),
            pltpu.VMEM_SHARED((n_pad,), jnp.float32),
        ],
        compiler_params=_SC_PARAMS,
    )
    def k(y_hbm, dis_hbm, row_hbm, col_hbm, z_hbm, s_hbm,
          row_v, col_v, gbuf, dbuf, zb2, zb1, sem_g, sem_s, z_sh, s_sh):
        cid = lax.axis_index("c")
        sid = lax.axis_index("s")
        wid = cid * NS + sid

        def zb2body(i, carry):
            for kk in range(d_hid // 16):
                zb2[i, pl.ds(kk * 16, 16)] = jnp.zeros((16,), jnp.float32)
            return carry

        lax.fori_loop(0, qs, zb2body, 0)

        def zb1body(i, carry):
            zb1[pl.ds(i * 16, 16)] = jnp.zeros((16,), jnp.float32)
            return carry

        lax.fori_loop(0, stripe // 16, zb1body, 0)
        for t in range(16):
            pltpu.sync_copy(zb2, z_sh.at[pl.ds(sid * stripe + t * qs, qs)])
        pltpu.sync_copy(zb1, s_sh.at[pl.ds(sid * stripe, stripe)])
        pltpu.sync_copy(row_hbm.at[wid], row_v)
        pltpu.sync_copy(col_hbm.at[wid], col_v)
        plsc.subcore_barrier()

        def body(j, carry):
            pltpu.sync_copy(y_hbm.at[row_v.at[j]], gbuf.at[0])
            pltpu.sync_copy(gbuf.at[0], z_sh.at[col_v.at[j]], add=True)
            pltpu.sync_copy(dis_hbm.at[col_v.at[j]], dbuf.at[0])
            pltpu.sync_copy(dbuf.at[0], s_sh.at[row_v.at[j]], add=True)
            return carry

        lax.fori_loop(0, cpt, body, 0)
        plsc.subcore_barrier()
        pltpu.sync_copy(
            z_sh.at[pl.ds(sid * stripe, stripe)],
            z_hbm.at[cid, pl.ds(sid * stripe, stripe)],
        )
        pltpu.sync_copy(
            s_sh.at[pl.ds(sid * stripe, stripe)],
            s_hbm.at[cid, pl.ds(sid * stripe, stripe)],
        )

    return k(y_pad, dis_pad, row_t, col_t)


def _final_tc(z0, z1, y, dis, s0, s1, mask, w2r, b1r, b2r, n, n_pad, bn):
    d_hid = y.shape[1]
    nblk = n_pad // bn

    def body(z0_ref, z1_ref, y_ref, dis_ref, s0_ref, s1_ref, m_ref,
             w2_ref, b1_ref, b2_ref, out_ref):
        i = pl.program_id(0)
        dis = dis_ref[...]
        pre = dis[:, None] * (z0_ref[...] + z1_ref[...] + y_ref[...]) + b1_ref[...]
        h1 = jnp.maximum(pre, 0.0)
        hw2 = jnp.sum(h1 * w2_ref[...], axis=1)
        s = s0_ref[...] + s1_ref[...] + dis
        part = jnp.sum(m_ref[...] * dis * s * hw2)

        @pl.when(i == 0)
        def _():
            out_ref[...] = jnp.zeros_like(out_ref)

        out_ref[...] = out_ref[...] + part

        @pl.when(i == nblk - 1)
        def _():
            out_ref[...] = out_ref[...] * (1.0 / n) + b2_ref[...]

    vec = lambda: pl.BlockSpec((bn,), lambda i: (i,))
    mat = lambda: pl.BlockSpec((bn, d_hid), lambda i: (i, 0))
    return pl.pallas_call(
        body,
        grid=(nblk,),
        in_specs=[
            mat(), mat(), mat(), vec(), vec(), vec(), vec(),
            pl.BlockSpec((1, d_hid), lambda i: (0, 0)),
            pl.BlockSpec((1, d_hid), lambda i: (0, 0)),
            pl.BlockSpec((1, 1), lambda i: (0, 0)),
        ],
        out_specs=pl.BlockSpec((1, 1), lambda i: (0, 0)),
        out_shape=jax.ShapeDtypeStruct((1, 1), jnp.float32),
    )(z0, z1, y, dis, s0, s1, mask, w2r, b1r, b2r)


def kernel(x, edge_index, W1, b1, W2, b2):
    n, d_in = x.shape
    e = edge_index.shape[1]
    d_hid = W1.shape[1]
    bn = 512
    kdepth = 1
    n_pad = math.ceil((n + 1) / (NS * bn // NS)) * bn
    cpt = math.ceil(e / (NW * CHUNK * kdepth)) * kdepth
    e_pad = NW * cpt * CHUNK

    row = edge_index[0]
    col = edge_index[1]
    pad_e = e_pad - e
    dummy = jnp.full((pad_e,), n, dtype=edge_index.dtype)
    row_t = jnp.concatenate([row, dummy]).reshape(NW, cpt, CHUNK)
    col_t = jnp.concatenate([col, dummy]).reshape(NW, cpt, CHUNK)
    x_pad = jnp.zeros((n_pad, d_in), x.dtype).at[:n].set(x)

    degp = _deg_hist(col_t, n_pad, cpt)
    y_pad, dis_pad = _prep_tc(x_pad, W1, degp[0], degp[1], n_pad, bn)
    z_p, s_p = _edge_pass(y_pad, dis_pad, row_t, col_t, n_pad, cpt, kdepth)

    mask = (jnp.arange(n_pad) < n).astype(jnp.float32)
    out2 = _final_tc(
        z_p[0], z_p[1], y_pad, dis_pad, s_p[0], s_p[1], mask,
        W2.reshape(1, d_hid), b1.reshape(1, d_hid), b2.reshape(1, 1),
        n, n_pad, bn,
    )
    return out2.reshape(1)


# speedup vs baseline: 2.0228x; 2.0228x over previous
import functools
import math

import jax
import jax.numpy as jnp
from jax import lax
from jax.experimental import pallas as pl
from jax.experimental.pallas import tpu as pltpu
from jax.experimental.pallas import tpu_sc as plsc

NC = 2
NS = 16
NW = NC * NS
CHUNK = 128


def _sc_mesh():
    return plsc.VectorSubcoreMesh(core_axis_name="c", subcore_axis_name="s")


_SC_PARAMS = pltpu.CompilerParams(
    use_tc_tiling_on_sc=False, needs_layout_passes=False)


def _deg_hist(col_t, n_pad, cpt):
    stripe = n_pad // NS

    @functools.partial(
        pl.kernel,
        out_type=jax.ShapeDtypeStruct((NC, n_pad), jnp.float32),
        mesh=_sc_mesh(),
        scratch_types=[
            pltpu.VMEM((cpt, CHUNK), jnp.int32),
            pltpu.VMEM((CHUNK,), jnp.float32),
            pltpu.VMEM((stripe,), jnp.float32),
            pltpu.VMEM_SHARED((n_pad,), jnp.float32),
        ],
        compiler_params=_SC_PARAMS,
    )
    def k(col_hbm, out_hbm, col_v, ones_v, zb_v, acc_sh):
        cid = lax.axis_index("c")
        sid = lax.axis_index("s")
        wid = cid * NS + sid
        for i in range(CHUNK // 16):
            ones_v[pl.ds(i * 16, 16)] = jnp.ones((16,), jnp.float32)

        def zb(i, carry):
            zb_v[pl.ds(i * 16, 16)] = jnp.zeros((16,), jnp.float32)
            return carry

        lax.fori_loop(0, stripe // 16, zb, 0)
        pltpu.sync_copy(zb_v, acc_sh.at[pl.ds(sid * stripe, stripe)])
        pltpu.sync_copy(col_hbm.at[wid], col_v)
        plsc.subcore_barrier()

        def body(j, carry):
            pltpu.sync_copy(ones_v, acc_sh.at[col_v.at[j]], add=True)
            return carry

        lax.fori_loop(0, cpt, body, 0)
        plsc.subcore_barrier()
        pltpu.sync_copy(
            acc_sh.at[pl.ds(sid * stripe, stripe)],
            out_hbm.at[cid, pl.ds(sid * stripe, stripe)],
        )

    return k(col_t)


def _prep_tc(x_pad, W1, d0, d1, n_pad, bn):
    d_in = x_pad.shape[1]
    d_hid = W1.shape[1]

    def body(x_ref, w_ref, d0_ref, d1_ref, y_ref, dis_ref):
        deg = d0_ref[...] + d1_ref[...] + 1.0
        dis = lax.rsqrt(deg)
        xw = jnp.dot(x_ref[...], w_ref[...], preferred_element_type=jnp.float32)
        y_ref[...] = xw * dis[:, None]
        dis_ref[...] = dis

    return pl.pallas_call(
        body,
        grid=(n_pad // bn,),
        in_specs=[
            pl.BlockSpec((bn, d_in), lambda i: (i, 0)),
            pl.BlockSpec((d_in, d_hid), lambda i: (0, 0)),
            pl.BlockSpec((bn,), lambda i: (i,)),
            pl.BlockSpec((bn,), lambda i: (i,)),
        ],
        out_specs=[
            pl.BlockSpec((bn, d_hid), lambda i: (i, 0)),
            pl.BlockSpec((bn,), lambda i: (i,)),
        ],
        out_shape=[
            jax.ShapeDtypeStruct((n_pad, d_hid), jnp.float32),
            jax.ShapeDtypeStruct((n_pad,), jnp.float32),
        ],
    )(x_pad, W1, d0, d1)


def _edge_pass(y_pad, dis_pad, row_t, col_t, n_pad, cpt, kdepth):
    stripe = n_pad // NS
    d_hid = y_pad.shape[1]
    qs = stripe // 16

    @functools.partial(
        pl.kernel,
        out_type=(
            jax.ShapeDtypeStruct((NC, n_pad, d_hid), jnp.float32),
            jax.ShapeDtypeStruct((NC, n_pad), jnp.float32),
        ),
        mesh=_sc_mesh(),
        scratch_types=[
            pltpu.VMEM((cpt, CHUNK), jnp.int32),
            pltpu.VMEM((cpt, CHUNK), jnp.int32),
            pltpu.VMEM((1, CHUNK, d_hid), jnp.float32),
            pltpu.VMEM((1, CHUNK), jnp.float32),
            pltpu.VMEM((qs, d_hid), jnp.float32),
            pltpu.VMEM((stripe,), jnp.float32),
            pltpu.VMEM((n_pad,), jnp.float32),
            pltpu.VMEM_SHARED((n_pad, d_hid), jnp.float32),
            pltpu.VMEM_SHARED((n_pad,), jnp.float32),
        ],
        compiler_params=_SC_PARAMS,
    )
    def k(y_hbm, dis_hbm, row_hbm, col_hbm, z_hbm, s_hbm,
          row_v, col_v, gbuf, dbuf, zb2, zb1, dis_v, z_sh, s_sh):
        cid = lax.axis_index("c")
        sid = lax.axis_index("s")
        wid = cid * NS + sid

        def zb2body(i, carry):
            for kk in range(d_hid // 16):
                zb2[i, pl.ds(kk * 16, 16)] = jnp.zeros((16,), jnp.float32)
            return carry

        lax.fori_loop(0, qs, zb2body, 0)

        def zb1body(i, carry):
            zb1[pl.ds(i * 16, 16)] = jnp.zeros((16,), jnp.float32)
            return carry

        lax.fori_loop(0, stripe // 16, zb1body, 0)
        for t in range(16):
            pltpu.sync_copy(zb2, z_sh.at[pl.ds(sid * stripe + t * qs, qs)])
        pltpu.sync_copy(zb1, s_sh.at[pl.ds(sid * stripe, stripe)])
        pltpu.sync_copy(row_hbm.at[wid], row_v)
        pltpu.sync_copy(col_hbm.at[wid], col_v)
        pltpu.sync_copy(dis_hbm, dis_v)
        plsc.subcore_barrier()

        def body(j, carry):
            pltpu.sync_copy(y_hbm.at[row_v.at[j]], gbuf.at[0])
            pltpu.sync_copy(gbuf.at[0], z_sh.at[col_v.at[j]], add=True)
            for kk in range(CHUNK // 16):
                idx = col_v[j, pl.ds(kk * 16, 16)]
                dbuf[0, pl.ds(kk * 16, 16)] = plsc.load_gather(dis_v, [idx])
            pltpu.sync_copy(dbuf.at[0], s_sh.at[row_v.at[j]], add=True)
            return carry

        lax.fori_loop(0, cpt, body, 0)
        plsc.subcore_barrier()
        pltpu.sync_copy(
            z_sh.at[pl.ds(sid * stripe, stripe)],
            z_hbm.at[cid, pl.ds(sid * stripe, stripe)],
        )
        pltpu.sync_copy(
            s_sh.at[pl.ds(sid * stripe, stripe)],
            s_hbm.at[cid, pl.ds(sid * stripe, stripe)],
        )

    return k(y_pad, dis_pad, row_t, col_t)


def _final_tc(z0, z1, y, dis, s0, s1, mask, w2r, b1r, b2r, n, n_pad, bn):
    d_hid = y.shape[1]
    nblk = n_pad // bn

    def body(z0_ref, z1_ref, y_ref, dis_ref, s0_ref, s1_ref, m_ref,
             w2_ref, b1_ref, b2_ref, out_ref):
        i = pl.program_id(0)
        dis = dis_ref[...]
        pre = dis[:, None] * (z0_ref[...] + z1_ref[...] + y_ref[...]) + b1_ref[...]
        h1 = jnp.maximum(pre, 0.0)
        hw2 = jnp.sum(h1 * w2_ref[...], axis=1)
        s = s0_ref[...] + s1_ref[...] + dis
        part = jnp.sum(m_ref[...] * dis * s * hw2)

        @pl.when(i == 0)
        def _():
            out_ref[...] = jnp.zeros_like(out_ref)

        out_ref[...] = out_ref[...] + part

        @pl.when(i == nblk - 1)
        def _():
            out_ref[...] = out_ref[...] * (1.0 / n) + b2_ref[...]

    vec = lambda: pl.BlockSpec((bn,), lambda i: (i,))
    mat = lambda: pl.BlockSpec((bn, d_hid), lambda i: (i, 0))
    return pl.pallas_call(
        body,
        grid=(nblk,),
        in_specs=[
            mat(), mat(), mat(), vec(), vec(), vec(), vec(),
            pl.BlockSpec((1, d_hid), lambda i: (0, 0)),
            pl.BlockSpec((1, d_hid), lambda i: (0, 0)),
            pl.BlockSpec((1, 1), lambda i: (0, 0)),
        ],
        out_specs=pl.BlockSpec((1, 1), lambda i: (0, 0)),
        out_shape=jax.ShapeDtypeStruct((1, 1), jnp.float32),
    )(z0, z1, y, dis, s0, s1, mask, w2r, b1r, b2r)


def kernel(x, edge_index, W1, b1, W2, b2):
    n, d_in = x.shape
    e = edge_index.shape[1]
    d_hid = W1.shape[1]
    bn = 512
    kdepth = 1
    n_pad = math.ceil((n + 1) / (NS * bn // NS)) * bn
    cpt = math.ceil(e / (NW * CHUNK * kdepth)) * kdepth
    e_pad = NW * cpt * CHUNK

    row = edge_index[0]
    col = edge_index[1]
    pad_e = e_pad - e
    spare = n_pad - n
    dummy = n + jnp.arange(pad_e, dtype=edge_index.dtype) % spare
    row_t = jnp.concatenate([row, dummy]).reshape(cpt, NW, CHUNK).swapaxes(0, 1)
    col_t = jnp.concatenate([col, dummy]).reshape(cpt, NW, CHUNK).swapaxes(0, 1)
    x_pad = jnp.zeros((n_pad, d_in), x.dtype).at[:n].set(x)

    degp = _deg_hist(col_t, n_pad, cpt)
    y_pad, dis_pad = _prep_tc(x_pad, W1, degp[0], degp[1], n_pad, bn)
    z_p, s_p = _edge_pass(y_pad, dis_pad, row_t, col_t, n_pad, cpt, kdepth)

    mask = (jnp.arange(n_pad) < n).astype(jnp.float32)
    out2 = _final_tc(
        z_p[0], z_p[1], y_pad, dis_pad, s_p[0], s_p[1], mask,
        W2.reshape(1, d_hid), b1.reshape(1, d_hid), b2.reshape(1, 1),
        n, n_pad, bn,
    )
    return out2.reshape(1)


# speedup vs baseline: 2.1218x; 1.0489x over previous
import functools
import math

import jax
import jax.numpy as jnp
from jax import lax
from jax.experimental import pallas as pl
from jax.experimental.pallas import tpu as pltpu
from jax.experimental.pallas import tpu_sc as plsc

NC = 2
NS = 16
NW = NC * NS
CHUNK = 128


def _sc_mesh():
    return plsc.VectorSubcoreMesh(core_axis_name="c", subcore_axis_name="s")


_SC_PARAMS = pltpu.CompilerParams(
    use_tc_tiling_on_sc=False, needs_layout_passes=False)


def _deg_hist(col_t, n_pad, cpt):
    stripe = n_pad // NS
    ept = cpt * CHUNK

    @functools.partial(
        pl.kernel,
        out_type=jax.ShapeDtypeStruct((NC, n_pad), jnp.float32),
        mesh=_sc_mesh(),
        scratch_types=[
            pltpu.VMEM((cpt, CHUNK), jnp.int32),
            pltpu.VMEM((ept,), jnp.int32),
            pltpu.VMEM((ept,), jnp.float32),
            pltpu.VMEM((stripe,), jnp.float32),
            pltpu.VMEM_SHARED((n_pad,), jnp.float32),
        ],
        compiler_params=_SC_PARAMS,
    )
    def k(col_hbm, out_hbm, col_v, cfl_v, ones_v, zb_v, acc_sh):
        cid = lax.axis_index("c")
        sid = lax.axis_index("s")
        wid = cid * NS + sid
        pltpu.sync_copy(col_hbm.at[wid], col_v)

        def ones_body(j, carry):
            for kk in range(CHUNK // 16):
                o = j * CHUNK + kk * 16
                ones_v[pl.ds(o, 16)] = jnp.ones((16,), jnp.float32)
                cfl_v[pl.ds(o, 16)] = col_v[j, pl.ds(kk * 16, 16)]
            return carry

        lax.fori_loop(0, cpt, ones_body, 0)

        def zb(i, carry):
            zb_v[pl.ds(i * 16, 16)] = jnp.zeros((16,), jnp.float32)
            return carry

        lax.fori_loop(0, stripe // 16, zb, 0)
        pltpu.sync_copy(zb_v, acc_sh.at[pl.ds(sid * stripe, stripe)])
        plsc.subcore_barrier()
        pltpu.sync_copy(ones_v, acc_sh.at[cfl_v], add=True)
        plsc.subcore_barrier()
        pltpu.sync_copy(
            acc_sh.at[pl.ds(sid * stripe, stripe)],
            out_hbm.at[cid, pl.ds(sid * stripe, stripe)],
        )

    return k(col_t)


def _prep_tc(x_pad, W1, d0, d1, n_pad, bn):
    d_in = x_pad.shape[1]
    d_hid = W1.shape[1]

    def body(x_ref, w_ref, d0_ref, d1_ref, y_ref, dis_ref):
        deg = d0_ref[...] + d1_ref[...] + 1.0
        dis = lax.rsqrt(deg)
        xw = jnp.dot(x_ref[...], w_ref[...], preferred_element_type=jnp.float32)
        y_ref[...] = xw * dis[:, None]
        dis_ref[...] = dis

    return pl.pallas_call(
        body,
        grid=(n_pad // bn,),
        in_specs=[
            pl.BlockSpec((bn, d_in), lambda i: (i, 0)),
            pl.BlockSpec((d_in, d_hid), lambda i: (0, 0)),
            pl.BlockSpec((bn,), lambda i: (i,)),
            pl.BlockSpec((bn,), lambda i: (i,)),
        ],
        out_specs=[
            pl.BlockSpec((bn, d_hid), lambda i: (i, 0)),
            pl.BlockSpec((bn,), lambda i: (i,)),
        ],
        out_shape=[
            jax.ShapeDtypeStruct((n_pad, d_hid), jnp.float32),
            jax.ShapeDtypeStruct((n_pad,), jnp.float32),
        ],
    )(x_pad, W1, d0, d1)


def _edge_pass(y_pad, dis_pad, row_t, col_t, n_pad, cpt, kdepth):
    stripe = n_pad // NS
    d_hid = y_pad.shape[1]
    qs = stripe // 16
    ept = cpt * CHUNK

    @functools.partial(
        pl.kernel,
        out_type=(
            jax.ShapeDtypeStruct((NC, n_pad, d_hid), jnp.float32),
            jax.ShapeDtypeStruct((NC, n_pad), jnp.float32),
        ),
        mesh=_sc_mesh(),
        scratch_types=[
            pltpu.VMEM((cpt, CHUNK), jnp.int32),
            pltpu.VMEM((cpt, CHUNK), jnp.int32),
            pltpu.VMEM((ept,), jnp.int32),
            pltpu.VMEM((1, CHUNK, d_hid), jnp.float32),
            pltpu.VMEM((ept,), jnp.float32),
            pltpu.VMEM((qs, d_hid), jnp.float32),
            pltpu.VMEM((stripe,), jnp.float32),
            pltpu.VMEM((n_pad,), jnp.float32),
            pltpu.VMEM_SHARED((n_pad, d_hid), jnp.float32),
            pltpu.VMEM_SHARED((n_pad,), jnp.float32),
        ],
        compiler_params=_SC_PARAMS,
    )
    def k(y_hbm, dis_hbm, row_hbm, col_hbm, z_hbm, s_hbm,
          row_v, col_v, rfl_v, gbuf, dbuf, zb2, zb1, dis_v, z_sh, s_sh):
        cid = lax.axis_index("c")
        sid = lax.axis_index("s")
        wid = cid * NS + sid

        def zb2body(i, carry):
            for kk in range(d_hid // 16):
                zb2[i, pl.ds(kk * 16, 16)] = jnp.zeros((16,), jnp.float32)
            return carry

        lax.fori_loop(0, qs, zb2body, 0)

        def zb1body(i, carry):
            zb1[pl.ds(i * 16, 16)] = jnp.zeros((16,), jnp.float32)
            return carry

        lax.fori_loop(0, stripe // 16, zb1body, 0)
        for t in range(16):
            pltpu.sync_copy(zb2, z_sh.at[pl.ds(sid * stripe + t * qs, qs)])
        pltpu.sync_copy(zb1, s_sh.at[pl.ds(sid * stripe, stripe)])
        pltpu.sync_copy(row_hbm.at[wid], row_v)
        pltpu.sync_copy(col_hbm.at[wid], col_v)
        pltpu.sync_copy(dis_hbm, dis_v)
        plsc.subcore_barrier()

        def body(j, carry):
            pltpu.sync_copy(y_hbm.at[row_v.at[j]], gbuf.at[0])
            pltpu.sync_copy(gbuf.at[0], z_sh.at[col_v.at[j]], add=True)
            for kk in range(CHUNK // 16):
                o = j * CHUNK + kk * 16
                idx = col_v[j, pl.ds(kk * 16, 16)]
                dbuf[pl.ds(o, 16)] = plsc.load_gather(dis_v, [idx])
                rfl_v[pl.ds(o, 16)] = row_v[j, pl.ds(kk * 16, 16)]
            return carry

        lax.fori_loop(0, cpt, body, 0)
        pltpu.sync_copy(dbuf, s_sh.at[rfl_v], add=True)
        plsc.subcore_barrier()
        pltpu.sync_copy(
            z_sh.at[pl.ds(sid * stripe, stripe)],
            z_hbm.at[cid, pl.ds(sid * stripe, stripe)],
        )
        pltpu.sync_copy(
            s_sh.at[pl.ds(sid * stripe, stripe)],
            s_hbm.at[cid, pl.ds(sid * stripe, stripe)],
        )

    return k(y_pad, dis_pad, row_t, col_t)


def _final_tc(z0, z1, y, dis, s0, s1, mask, w2r, b1r, b2r, n, n_pad, bn):
    d_hid = y.shape[1]
    nblk = n_pad // bn

    def body(z0_ref, z1_ref, y_ref, dis_ref, s0_ref, s1_ref, m_ref,
             w2_ref, b1_ref, b2_ref, out_ref):
        i = pl.program_id(0)
        dis = dis_ref[...]
        pre = dis[:, None] * (z0_ref[...] + z1_ref[...] + y_ref[...]) + b1_ref[...]
        h1 = jnp.maximum(pre, 0.0)
        hw2 = jnp.sum(h1 * w2_ref[...], axis=1)
        s = s0_ref[...] + s1_ref[...] + dis
        part = jnp.sum(m_ref[...] * dis * s * hw2)

        @pl.when(i == 0)
        def _():
            out_ref[...] = jnp.zeros_like(out_ref)

        out_ref[...] = out_ref[...] + part

        @pl.when(i == nblk - 1)
        def _():
            out_ref[...] = out_ref[...] * (1.0 / n) + b2_ref[...]

    vec = lambda: pl.BlockSpec((bn,), lambda i: (i,))
    mat = lambda: pl.BlockSpec((bn, d_hid), lambda i: (i, 0))
    return pl.pallas_call(
        body,
        grid=(nblk,),
        in_specs=[
            mat(), mat(), mat(), vec(), vec(), vec(), vec(),
            pl.BlockSpec((1, d_hid), lambda i: (0, 0)),
            pl.BlockSpec((1, d_hid), lambda i: (0, 0)),
            pl.BlockSpec((1, 1), lambda i: (0, 0)),
        ],
        out_specs=pl.BlockSpec((1, 1), lambda i: (0, 0)),
        out_shape=jax.ShapeDtypeStruct((1, 1), jnp.float32),
    )(z0, z1, y, dis, s0, s1, mask, w2r, b1r, b2r)


def kernel(x, edge_index, W1, b1, W2, b2):
    n, d_in = x.shape
    e = edge_index.shape[1]
    d_hid = W1.shape[1]
    bn = 512
    kdepth = 1
    n_pad = math.ceil((n + 1) / (NS * bn // NS)) * bn
    cpt = math.ceil(e / (NW * CHUNK * kdepth)) * kdepth
    e_pad = NW * cpt * CHUNK

    row = edge_index[0]
    col = edge_index[1]
    pad_e = e_pad - e
    spare = n_pad - n
    dummy = n + jnp.arange(pad_e, dtype=edge_index.dtype) % spare
    row_t = jnp.concatenate([row, dummy]).reshape(cpt, NW, CHUNK).swapaxes(0, 1)
    col_t = jnp.concatenate([col, dummy]).reshape(cpt, NW, CHUNK).swapaxes(0, 1)
    x_pad = jnp.zeros((n_pad, d_in), x.dtype).at[:n].set(x)

    degp = _deg_hist(col_t, n_pad, cpt)
    y_pad, dis_pad = _prep_tc(x_pad, W1, degp[0], degp[1], n_pad, bn)
    z_p, s_p = _edge_pass(y_pad, dis_pad, row_t, col_t, n_pad, cpt, kdepth)

    mask = (jnp.arange(n_pad) < n).astype(jnp.float32)
    out2 = _final_tc(
        z_p[0], z_p[1], y_pad, dis_pad, s_p[0], s_p[1], mask,
        W2.reshape(1, d_hid), b1.reshape(1, d_hid), b2.reshape(1, 1),
        n, n_pad, bn,
    )
    return out2.reshape(1)
